# baseline (speedup 1.0000x reference)
import jax
import jax.numpy as jnp
from jax.experimental import pallas as pl


def _att_map_body(ego_ref, nb_ref, out_ref):
    out_ref[...] = (1.0 - ego_ref[...]) * nb_ref[...]


def kernel(ego_conf, nb_conf, delta=0.25):
    h = ego_conf.shape[-1]
    att = pl.pallas_call(
        _att_map_body,
        out_shape=jax.ShapeDtypeStruct(ego_conf.shape, jnp.float32),
    )(ego_conf, nb_conf)
    K = att.size // 4
    _, indices = jax.lax.top_k(att.reshape(-1), K)
    return (indices // h, indices % h)


# speedup vs baseline: 1.0067x; 1.0067x over previous
import functools

import jax
import jax.numpy as jnp
from jax import lax
from jax.experimental import pallas as pl
from jax.experimental.pallas import tpu as pltpu
from jax.experimental.pallas import tpu_sc as plsc

N = 262144
H = 512
K = N // 4
T = 16
C = N // T
NV = C // 16
BINS = 1024
GRID = BINS * T
KPT = K // T


def _iota():
    return lax.iota(jnp.int32, 16)


def _sc_body(ego_hbm, nb_hbm, rows_hbm, cols_hbm,
             key_loc, nb_loc, idx_loc, pos_loc, grid_loc,
             hist_loc, gidx_loc, oidx_loc, orow_loc, ocol_loc,
             akey, aidx, grid_sh,
             sem0, sem1, sem2, sem3):
    cid = lax.axis_index("c")
    sid = lax.axis_index("s")

    @pl.when(cid == 0)
    def _():
        wid = sid
        base = wid * C

        pltpu.sync_copy(ego_hbm.at[pl.ds(base, C)], key_loc)
        pltpu.sync_copy(nb_hbm.at[pl.ds(base, C)], nb_loc)

        def p0(j, _):
            e = key_loc[pl.ds(j * 16, 16)]
            nbv = nb_loc[pl.ds(j * 16, 16)]
            key_loc[pl.ds(j * 16, 16)] = (1.0 - e) * nbv
            idx_loc[pl.ds(j * 16, 16)] = base + j * 16 + _iota()
            return 0

        lax.fori_loop(0, NV, p0, 0)

        def p_gidx(j, _):
            gidx_loc[pl.ds(j * 16, 16)] = (j * 16 + _iota()) * T + wid
            return 0

        lax.fori_loop(0, BINS // 16, p_gidx, 0)

        for p, shift in enumerate((0, 10, 20)):
            if p > 0:
                def p_lin(j, _):
                    pos_loc[pl.ds(j * 16, 16)] = base + j * 16 + _iota()
                    return 0

                lax.fori_loop(0, NV, p_lin, 0)
                cpk = pltpu.async_copy(akey.at[pos_loc], key_loc, sem0)
                cpi = pltpu.async_copy(aidx.at[pos_loc], idx_loc, sem1)
                cpk.wait()
                cpi.wait()
                plsc.subcore_barrier()

            def p_zero(j, _):
                hist_loc[pl.ds(j * 16, 16)] = jnp.zeros((16,), jnp.int32)
                return 0

            lax.fori_loop(0, BINS // 16, p_zero, 0)

            def p_hist(j, _):
                k32 = plsc.bitcast(key_loc[pl.ds(j * 16, 16)], jnp.int32)
                d = 1023 - ((k32 >> shift) & 1023)
                plsc.addupdate_scatter(hist_loc, [d], jnp.ones((16,), jnp.int32))
                return 0

            lax.fori_loop(0, NV, p_hist, 0)

            pltpu.async_copy(hist_loc, grid_sh.at[gidx_loc], sem2).wait()
            plsc.subcore_barrier()

            pltpu.sync_copy(grid_sh, grid_loc)

            def p_scan(j, carry):
                v = grid_loc[pl.ds(j * 16, 16)]
                inc = plsc.cumsum(v)
                grid_loc[pl.ds(j * 16, 16)] = inc - v + carry
                return carry + jnp.sum(v)

            lax.fori_loop(0, GRID // 16, p_scan, jnp.int32(0))
            plsc.subcore_barrier()

            def p_perm(j, _):
                k32 = plsc.bitcast(key_loc[pl.ds(j * 16, 16)], jnp.int32)
                d = 1023 - ((k32 >> shift) & 1023)
                gi = d * T + wid
                off = plsc.load_gather(grid_loc, [gi])
                cnt, lastm = plsc.scan_count(gi)
                pos_loc[pl.ds(j * 16, 16)] = off + cnt - 1
                plsc.addupdate_scatter(grid_loc, [gi], cnt, mask=lastm)
                return 0

            lax.fori_loop(0, NV, p_perm, 0)

            if p < 2:
                cpk = pltpu.async_copy(key_loc, akey.at[pos_loc], sem0)
                cpi = pltpu.async_copy(idx_loc, aidx.at[pos_loc], sem1)
                cpk.wait()
                cpi.wait()
            else:
                pltpu.async_copy(idx_loc, aidx.at[pos_loc], sem1).wait()
            plsc.subcore_barrier()

        obase = wid * KPT

        def p_oidx(j, _):
            oidx_loc[pl.ds(j * 16, 16)] = obase + j * 16 + _iota()
            return 0

        lax.fori_loop(0, KPT // 16, p_oidx, 0)
        pltpu.async_copy(aidx.at[oidx_loc], oidx_loc, sem3).wait()

        def p_out(j, _):
            v = oidx_loc[pl.ds(j * 16, 16)]
            orow_loc[pl.ds(j * 16, 16)] = lax.shift_right_logical(v, 9)
            ocol_loc[pl.ds(j * 16, 16)] = v & (H - 1)
            return 0

        lax.fori_loop(0, KPT // 16, p_out, 0)
        pltpu.sync_copy(orow_loc, rows_hbm.at[pl.ds(obase, KPT)])
        pltpu.sync_copy(ocol_loc, cols_hbm.at[pl.ds(obase, KPT)])


@functools.partial(jax.jit, static_argnames=())
def _run(ego_flat, nb_flat):
    mesh = plsc.VectorSubcoreMesh(core_axis_name="c", subcore_axis_name="s")
    f = pl.kernel(
        _sc_body,
        out_type=(jax.ShapeDtypeStruct((K,), jnp.int32),
                  jax.ShapeDtypeStruct((K,), jnp.int32)),
        mesh=mesh,
        compiler_params=pltpu.CompilerParams(needs_layout_passes=False),
        scratch_types=[
            pltpu.VMEM((C,), jnp.float32),
            pltpu.VMEM((C,), jnp.float32),
            pltpu.VMEM((C,), jnp.int32),
            pltpu.VMEM((C,), jnp.int32),
            pltpu.VMEM((GRID,), jnp.int32),
            pltpu.VMEM((BINS,), jnp.int32),
            pltpu.VMEM((BINS,), jnp.int32),
            pltpu.VMEM((KPT,), jnp.int32),
            pltpu.VMEM((KPT,), jnp.int32),
            pltpu.VMEM((KPT,), jnp.int32),
            pltpu.VMEM_SHARED((N,), jnp.float32),
            pltpu.VMEM_SHARED((N,), jnp.int32),
            pltpu.VMEM_SHARED((GRID,), jnp.int32),
            pltpu.SemaphoreType.DMA,
            pltpu.SemaphoreType.DMA,
            pltpu.SemaphoreType.DMA,
            pltpu.SemaphoreType.DMA,
        ],
    )
    return f(ego_flat, nb_flat)


def kernel(ego_conf, nb_conf, delta=0.25):
    del delta
    rows_idx, cols_idx = _run(ego_conf.reshape(-1), nb_conf.reshape(-1))
    return (rows_idx, cols_idx)


# speedup vs baseline: 1.6116x; 1.6010x over previous
import functools

import jax
import jax.numpy as jnp
from jax import lax
from jax.experimental import pallas as pl
from jax.experimental.pallas import tpu as pltpu
from jax.experimental.pallas import tpu_sc as plsc

N = 262144
H = 512
K = N // 4
T = 16
C = N // T
NV = C // 16
BINS = 1024
GRID = BINS * T
SLICE = GRID // T
KPT = K // T
HCOPIES = 4


def _iota():
    return lax.iota(jnp.int32, 16)


def _sc_body(ego_hbm, nb_hbm, rows_hbm, cols_hbm,
             key_loc, nb_loc, idx_loc, pos_loc,
             hist_loc, off_loc, gidx_loc, sidx_loc, slc_loc, tot_loc,
             akey, aidx, grid_sh, tot_sh,
             sem0, sem1, sem2, sem3):
    cid = lax.axis_index("c")
    sid = lax.axis_index("s")

    @pl.when(cid == 0)
    def _():
        wid = sid
        base = wid * C

        pltpu.sync_copy(ego_hbm.at[pl.ds(base, C)], key_loc)
        pltpu.sync_copy(nb_hbm.at[pl.ds(base, C)], nb_loc)

        @plsc.parallel_loop(0, NV, unroll=4)
        def _(j):
            e = key_loc[pl.ds(j * 16, 16)]
            nbv = nb_loc[pl.ds(j * 16, 16)]
            key_loc[pl.ds(j * 16, 16)] = (1.0 - e) * nbv
            idx_loc[pl.ds(j * 16, 16)] = base + j * 16 + _iota()

        @plsc.parallel_loop(0, BINS // 16, unroll=4)
        def _(j):
            gidx_loc[pl.ds(j * 16, 16)] = (j * 16 + _iota()) * T + wid
            sidx_loc[pl.ds(j * 16, 16)] = wid * SLICE + j * 16 + _iota()

        for p, shift in enumerate((0, 10, 20)):
            if p > 0:
                @plsc.parallel_loop(0, NV, unroll=4)
                def _(j):
                    pos_loc[pl.ds(j * 16, 16)] = base + j * 16 + _iota()

                cpk = pltpu.async_copy(akey.at[pos_loc], key_loc, sem0)
                cpi = pltpu.async_copy(aidx.at[pos_loc], idx_loc, sem1)
                cpk.wait()
                cpi.wait()

            @plsc.parallel_loop(0, HCOPIES * BINS // 16, unroll=4)
            def _(j):
                hist_loc[pl.ds(j * 16, 16)] = jnp.zeros((16,), jnp.int32)

            @plsc.parallel_loop(0, NV, unroll=HCOPIES)
            def _(j):
                k32 = plsc.bitcast(key_loc[pl.ds(j * 16, 16)], jnp.int32)
                d = (1023 - ((k32 >> shift) & 1023)) + (j % HCOPIES) * BINS
                plsc.addupdate_scatter(hist_loc, [d],
                                       jnp.ones((16,), jnp.int32))

            @plsc.parallel_loop(0, BINS // 16, unroll=4)
            def _(j):
                acc = hist_loc[pl.ds(j * 16, 16)]
                for c in range(1, HCOPIES):
                    acc = acc + hist_loc[pl.ds(c * BINS + j * 16, 16)]
                hist_loc[pl.ds(j * 16, 16)] = acc

            pltpu.async_copy(hist_loc.at[pl.ds(0, BINS)],
                             grid_sh.at[gidx_loc], sem2).wait()
            plsc.subcore_barrier()

            pltpu.async_copy(grid_sh.at[sidx_loc], slc_loc, sem2).wait()

            def p_scan(j, carry):
                v = slc_loc[pl.ds(j * 16, 16)]
                inc = plsc.cumsum(v)
                slc_loc[pl.ds(j * 16, 16)] = inc - v + carry
                return carry + jnp.sum(v)

            total = lax.fori_loop(0, SLICE // 16, p_scan, jnp.int32(0))
            tot_loc[pl.ds(0, 16)] = jnp.where(_iota() == 0, total, 0)
            ti = jnp.where(_iota() == 0, wid, T + wid)
            cps = pltpu.async_copy(slc_loc, grid_sh.at[sidx_loc], sem2)
            cpt = pltpu.async_copy(tot_loc.at[pl.ds(0, 16)], tot_sh.at[ti], sem3)
            cps.wait()
            cpt.wait()
            plsc.subcore_barrier()

            pltpu.sync_copy(tot_sh, tot_loc)
            cpo = pltpu.async_copy(grid_sh.at[gidx_loc], off_loc, sem2)
            t16 = tot_loc[pl.ds(0, 16)]
            ctot = plsc.cumsum(t16) - t16
            tot_loc[pl.ds(0, 16)] = ctot
            cpo.wait()

            @plsc.parallel_loop(0, BINS // 16, unroll=4)
            def _(j):
                gi = (j * 16 + _iota()) * T + wid
                carry = plsc.load_gather(tot_loc, [gi >> 10])
                off_loc[pl.ds(j * 16, 16)] = off_loc[pl.ds(j * 16, 16)] + carry

            @plsc.parallel_loop(0, NV, unroll=4)
            def _(j):
                k32 = plsc.bitcast(key_loc[pl.ds(j * 16, 16)], jnp.int32)
                d = 1023 - ((k32 >> shift) & 1023)
                cnt, _unused = plsc.scan_count(d)
                nb_loc[pl.ds(j * 16, 16)] = plsc.bitcast(cnt, jnp.float32)

            def p_perm(j, _):
                k32 = plsc.bitcast(key_loc[pl.ds(j * 16, 16)], jnp.int32)
                d = 1023 - ((k32 >> shift) & 1023)
                off = plsc.load_gather(off_loc, [d])
                cnt = plsc.bitcast(nb_loc[pl.ds(j * 16, 16)], jnp.int32)
                pos_loc[pl.ds(j * 16, 16)] = off + cnt - 1
                plsc.addupdate_scatter(off_loc, [d], jnp.ones((16,), jnp.int32))
                return 0

            lax.fori_loop(0, NV, p_perm, 0)

            if p < 2:
                cpk = pltpu.async_copy(key_loc, akey.at[pos_loc], sem0)
                cpi = pltpu.async_copy(idx_loc, aidx.at[pos_loc], sem1)
                cpk.wait()
                cpi.wait()
            else:
                pltpu.async_copy(idx_loc, aidx.at[pos_loc], sem1).wait()
            plsc.subcore_barrier()

        obase = wid * KPT

        @plsc.parallel_loop(0, KPT // 16, unroll=4)
        def _(j):
            pos_loc[pl.ds(j * 16, 16)] = obase + j * 16 + _iota()

        pltpu.async_copy(aidx.at[pos_loc.at[pl.ds(0, KPT)]],
                         idx_loc.at[pl.ds(0, KPT)], sem3).wait()

        @plsc.parallel_loop(0, KPT // 16, unroll=4)
        def _(j):
            v = idx_loc[pl.ds(j * 16, 16)]
            idx_loc[pl.ds(j * 16, 16)] = lax.shift_right_logical(v, 9)
            pos_loc[pl.ds(j * 16, 16)] = v & (H - 1)

        pltpu.sync_copy(idx_loc.at[pl.ds(0, KPT)], rows_hbm.at[pl.ds(obase, KPT)])
        pltpu.sync_copy(pos_loc.at[pl.ds(0, KPT)], cols_hbm.at[pl.ds(obase, KPT)])


@jax.jit
def _run(ego_flat, nb_flat):
    mesh = plsc.VectorSubcoreMesh(core_axis_name="c", subcore_axis_name="s")
    f = pl.kernel(
        _sc_body,
        out_type=(jax.ShapeDtypeStruct((K,), jnp.int32),
                  jax.ShapeDtypeStruct((K,), jnp.int32)),
        mesh=mesh,
        compiler_params=pltpu.CompilerParams(needs_layout_passes=False),
        scratch_types=[
            pltpu.VMEM((C,), jnp.float32),
            pltpu.VMEM((C,), jnp.float32),
            pltpu.VMEM((C,), jnp.int32),
            pltpu.VMEM((C,), jnp.int32),
            pltpu.VMEM((HCOPIES * BINS,), jnp.int32),
            pltpu.VMEM((BINS,), jnp.int32),
            pltpu.VMEM((BINS,), jnp.int32),
            pltpu.VMEM((SLICE,), jnp.int32),
            pltpu.VMEM((SLICE,), jnp.int32),
            pltpu.VMEM((2 * T,), jnp.int32),
            pltpu.VMEM_SHARED((N,), jnp.float32),
            pltpu.VMEM_SHARED((N,), jnp.int32),
            pltpu.VMEM_SHARED((GRID,), jnp.int32),
            pltpu.VMEM_SHARED((2 * T,), jnp.int32),
            pltpu.SemaphoreType.DMA,
            pltpu.SemaphoreType.DMA,
            pltpu.SemaphoreType.DMA,
            pltpu.SemaphoreType.DMA,
        ],
    )
    return f(ego_flat, nb_flat)


def kernel(ego_conf, nb_conf, delta=0.25):
    del delta
    rows_idx, cols_idx = _run(ego_conf.reshape(-1), nb_conf.reshape(-1))
    return (rows_idx, cols_idx)


# speedup vs baseline: 1.6431x; 1.0195x over previous
import functools

import jax
import jax.numpy as jnp
from jax import lax
from jax.experimental import pallas as pl
from jax.experimental.pallas import tpu as pltpu
from jax.experimental.pallas import tpu_sc as plsc

N = 262144
H = 512
K = N // 4
T = 16
C = N // T
NV = C // 16
BINS = 1024
GRID = BINS * T
SLICE = GRID // T
KPT = K // T
HCOPIES = 8


def _iota():
    return lax.iota(jnp.int32, 16)


def _sc_body(ego_hbm, nb_hbm, rows_hbm, cols_hbm,
             key_loc, nb_loc, idx_loc, pos_loc,
             hist_loc, off_loc, gidx_loc, sidx_loc, slc_loc, tot_loc,
             akey, aidx, grid_sh, tot_sh,
             sem0, sem1, sem2, sem3):
    cid = lax.axis_index("c")
    sid = lax.axis_index("s")

    @pl.when(cid == 0)
    def _():
        wid = sid
        base = wid * C

        pltpu.sync_copy(ego_hbm.at[pl.ds(base, C)], key_loc)
        pltpu.sync_copy(nb_hbm.at[pl.ds(base, C)], nb_loc)

        @plsc.parallel_loop(0, NV, unroll=4)
        def _(j):
            e = key_loc[pl.ds(j * 16, 16)]
            nbv = nb_loc[pl.ds(j * 16, 16)]
            key_loc[pl.ds(j * 16, 16)] = (1.0 - e) * nbv
            idx_loc[pl.ds(j * 16, 16)] = base + j * 16 + _iota()

        @plsc.parallel_loop(0, BINS // 16, unroll=4)
        def _(j):
            gidx_loc[pl.ds(j * 16, 16)] = (j * 16 + _iota()) * T + wid
            sidx_loc[pl.ds(j * 16, 16)] = wid * SLICE + j * 16 + _iota()

        for p, shift in enumerate((0, 10, 20)):
            if p > 0:
                @plsc.parallel_loop(0, NV, unroll=4)
                def _(j):
                    pos_loc[pl.ds(j * 16, 16)] = base + j * 16 + _iota()

                cpk = pltpu.async_copy(akey.at[pos_loc], key_loc, sem0)
                cpi = pltpu.async_copy(aidx.at[pos_loc], idx_loc, sem1)
                cpk.wait()
                cpi.wait()

            @plsc.parallel_loop(0, HCOPIES * BINS // 16, unroll=8)
            def _(j):
                hist_loc[pl.ds(j * 16, 16)] = jnp.zeros((16,), jnp.int32)

            @plsc.parallel_loop(0, NV, unroll=HCOPIES)
            def _(j):
                k32 = plsc.bitcast(key_loc[pl.ds(j * 16, 16)], jnp.int32)
                d = 1023 - ((k32 >> shift) & 1023)
                cnt, _unused = plsc.scan_count(d)
                nb_loc[pl.ds(j * 16, 16)] = plsc.bitcast(cnt, jnp.float32)
                plsc.addupdate_scatter(hist_loc, [d + (j % HCOPIES) * BINS],
                                       jnp.ones((16,), jnp.int32))

            @plsc.parallel_loop(0, BINS // 16, unroll=4)
            def _(j):
                acc = hist_loc[pl.ds(j * 16, 16)]
                for c in range(1, HCOPIES):
                    acc = acc + hist_loc[pl.ds(c * BINS + j * 16, 16)]
                hist_loc[pl.ds(j * 16, 16)] = acc

            pltpu.async_copy(hist_loc.at[pl.ds(0, BINS)],
                             grid_sh.at[gidx_loc], sem2).wait()
            plsc.subcore_barrier()

            pltpu.async_copy(grid_sh.at[sidx_loc], slc_loc, sem2).wait()

            def p_scan(j, carry):
                v = slc_loc[pl.ds(j * 16, 16)]
                inc = plsc.cumsum(v)
                slc_loc[pl.ds(j * 16, 16)] = inc - v + carry
                return carry + jnp.sum(v)

            total = lax.fori_loop(0, SLICE // 16, p_scan, jnp.int32(0))
            tot_loc[pl.ds(0, 16)] = jnp.where(_iota() == 0, total, 0)
            ti = jnp.where(_iota() == 0, wid, T + wid)
            cps = pltpu.async_copy(slc_loc, grid_sh.at[sidx_loc], sem2)
            cpt = pltpu.async_copy(tot_loc.at[pl.ds(0, 16)], tot_sh.at[ti], sem3)
            cps.wait()
            cpt.wait()
            plsc.subcore_barrier()

            pltpu.sync_copy(tot_sh, tot_loc)
            cpo = pltpu.async_copy(grid_sh.at[gidx_loc], off_loc, sem2)
            t16 = tot_loc[pl.ds(0, 16)]
            ctot = plsc.cumsum(t16) - t16
            tot_loc[pl.ds(0, 16)] = ctot
            cpo.wait()

            @plsc.parallel_loop(0, BINS // 16, unroll=4)
            def _(j):
                gi = (j * 16 + _iota()) * T + wid
                carry = plsc.load_gather(tot_loc, [gi >> 10])
                off_loc[pl.ds(j * 16, 16)] = off_loc[pl.ds(j * 16, 16)] + carry

            def p_perm(j, _):
                k32 = plsc.bitcast(key_loc[pl.ds(j * 16, 16)], jnp.int32)
                d = 1023 - ((k32 >> shift) & 1023)
                off = plsc.load_gather(off_loc, [d])
                cnt = plsc.bitcast(nb_loc[pl.ds(j * 16, 16)], jnp.int32)
                pos_loc[pl.ds(j * 16, 16)] = off + cnt - 1
                plsc.addupdate_scatter(off_loc, [d], jnp.ones((16,), jnp.int32))
                return 0

            lax.fori_loop(0, NV, p_perm, 0)

            if p < 2:
                cpk = pltpu.async_copy(key_loc, akey.at[pos_loc], sem0)
                cpi = pltpu.async_copy(idx_loc, aidx.at[pos_loc], sem1)
                cpk.wait()
                cpi.wait()
            else:
                pltpu.async_copy(idx_loc, aidx.at[pos_loc], sem1).wait()
            plsc.subcore_barrier()

        obase = wid * KPT

        @plsc.parallel_loop(0, KPT // 16, unroll=4)
        def _(j):
            pos_loc[pl.ds(j * 16, 16)] = obase + j * 16 + _iota()

        pltpu.async_copy(aidx.at[pos_loc.at[pl.ds(0, KPT)]],
                         idx_loc.at[pl.ds(0, KPT)], sem3).wait()

        @plsc.parallel_loop(0, KPT // 16, unroll=4)
        def _(j):
            v = idx_loc[pl.ds(j * 16, 16)]
            idx_loc[pl.ds(j * 16, 16)] = lax.shift_right_logical(v, 9)
            pos_loc[pl.ds(j * 16, 16)] = v & (H - 1)

        pltpu.sync_copy(idx_loc.at[pl.ds(0, KPT)], rows_hbm.at[pl.ds(obase, KPT)])
        pltpu.sync_copy(pos_loc.at[pl.ds(0, KPT)], cols_hbm.at[pl.ds(obase, KPT)])


@jax.jit
def _run(ego_flat, nb_flat):
    mesh = plsc.VectorSubcoreMesh(core_axis_name="c", subcore_axis_name="s")
    f = pl.kernel(
        _sc_body,
        out_type=(jax.ShapeDtypeStruct((K,), jnp.int32),
                  jax.ShapeDtypeStruct((K,), jnp.int32)),
        mesh=mesh,
        compiler_params=pltpu.CompilerParams(needs_layout_passes=False),
        scratch_types=[
            pltpu.VMEM((C,), jnp.float32),
            pltpu.VMEM((C,), jnp.float32),
            pltpu.VMEM((C,), jnp.int32),
            pltpu.VMEM((C,), jnp.int32),
            pltpu.VMEM((HCOPIES * BINS,), jnp.int32),
            pltpu.VMEM((BINS,), jnp.int32),
            pltpu.VMEM((BINS,), jnp.int32),
            pltpu.VMEM((SLICE,), jnp.int32),
            pltpu.VMEM((SLICE,), jnp.int32),
            pltpu.VMEM((2 * T,), jnp.int32),
            pltpu.VMEM_SHARED((N,), jnp.float32),
            pltpu.VMEM_SHARED((N,), jnp.int32),
            pltpu.VMEM_SHARED((GRID,), jnp.int32),
            pltpu.VMEM_SHARED((2 * T,), jnp.int32),
            pltpu.SemaphoreType.DMA,
            pltpu.SemaphoreType.DMA,
            pltpu.SemaphoreType.DMA,
            pltpu.SemaphoreType.DMA,
        ],
    )
    return f(ego_flat, nb_flat)


def kernel(ego_conf, nb_conf, delta=0.25):
    del delta
    rows_idx, cols_idx = _run(ego_conf.reshape(-1), nb_conf.reshape(-1))
    return (rows_idx, cols_idx)
